# baseline (speedup 1.0000x reference)
import functools

import jax
import jax.numpy as jnp
from jax import lax
from jax.experimental import pallas as pl
from jax.experimental.pallas import tpu as pltpu
from jax.experimental.pallas import tpu_sc as plsc

NW = 32
CHUNK = 4096
NBUFS = 4
SAMP = 4096


def _sc_body(N, C, SEGS, NBUF, SAMP_PAD, samp_hbm, idx_hbm, mask_hbm, pos_hbm,
             out_hbm, samp_v, *scr):
    bufs = tuple((scr[4 * i], scr[4 * i + 1], scr[4 * i + 2], scr[4 * i + 3])
                 for i in range(NBUF))
    acc_e, acc_x, acc_y, out_v = scr[4 * NBUF:4 * NBUF + 4]
    cid = lax.axis_index("c")
    sid = lax.axis_index("s")
    w = sid * 2 + cid
    lane = lax.iota(jnp.int32, 16)
    zf = jnp.zeros((16,), jnp.float32)

    def zero_body(j, _):
        acc_e[pl.ds(j * 16, 16)] = zf
        acc_x[pl.ds(j * 16, 16)] = zf
        acc_y[pl.ds(j * 16, 16)] = zf
        return 0

    lax.fori_loop(0, SEGS // 16, zero_body, 0)
    for i in range(NBUF):
        bufs[i][0][pl.ds(C, 16)] = jnp.zeros((16,), jnp.int32)

    g_lo = w * SEGS
    g_hi = g_lo + SEGS

    pltpu.sync_copy(samp_hbm, samp_v)
    zi = jnp.zeros((16,), jnp.int32)

    def cnt_body(j, carry):
        c1, c2 = carry
        v = samp_v[pl.ds(j * 16, 16)]
        c1 = c1 + jnp.where(v < g_lo, 1, 0)
        c2 = c2 + jnp.where(v < g_hi, 1, 0)
        return (c1, c2)

    c1v, c2v = lax.fori_loop(0, SAMP_PAD // 16, cnt_body, (zi, zi), unroll=4)
    cnt1 = jnp.sum(c1v)
    cnt2 = jnp.sum(c2v)
    a_lo = jnp.maximum(cnt1 - 1, 0) * SAMP
    e_lo = a_lo
    e_hi = jnp.minimum(cnt2 * SAMP, N)
    n_chunks = (e_hi - a_lo + (C - 1)) // C

    def chunk_w0(k):
        return pl.multiple_of(jnp.minimum(a_lo + k * C, N - C), 128)

    def start_chunk(k, b):
        iv, mv, pv, sem = bufs[b]
        w0 = chunk_w0(k)
        pltpu.async_copy(idx_hbm.at[pl.ds(w0, C)], iv.at[pl.ds(0, C)], sem)
        pltpu.async_copy(mask_hbm.at[pl.ds(w0, C)], mv, sem)
        pltpu.async_copy(pos_hbm.at[pl.ds(pl.multiple_of(2 * w0, 256), 2 * C)],
                         pv, sem)

    def wait_chunk(b):
        iv, mv, pv, sem = bufs[b]
        pltpu.make_async_copy(idx_hbm.at[pl.ds(0, C)], iv.at[pl.ds(0, C)],
                              sem).wait()
        pltpu.make_async_copy(mask_hbm.at[pl.ds(0, C)], mv, sem).wait()
        pltpu.make_async_copy(pos_hbm.at[pl.ds(0, 2 * C)], pv, sem).wait()

    last = lane == 15
    notlast = jnp.logical_not(last)

    def process_chunk(k, b):
        iv, mv, pv, _ = bufs[b]
        c0 = a_lo + k * C
        w0 = chunk_w0(k)
        clean = (c0 >= e_lo + SAMP) & (c0 + C <= e_hi - SAMP)

        def scat(li, li1, ce, cx, cy, m_p, m_m):
            plsc.addupdate_scatter(acc_e, [li], ce, mask=m_p)
            plsc.addupdate_scatter(acc_x, [li], cx, mask=m_p)
            plsc.addupdate_scatter(acc_y, [li], cy, mask=m_p)
            plsc.addupdate_scatter(acc_e, [li1], -ce, mask=m_m)
            plsc.addupdate_scatter(acc_x, [li1], -cx, mask=m_m)
            plsc.addupdate_scatter(acc_y, [li1], -cy, mask=m_m)

        def vec_fast(j, _):
            ii = j * 16
            vi = iv[pl.ds(ii, 16)]
            vi1 = iv[pl.ds(ii + 1, 16)]
            e = jnp.exp(mv[pl.ds(ii, 16)])
            xb = 2 * ii - (ii % 128)
            x = pv[pl.ds(xb, 16)]
            y = pv[pl.ds(xb + 128, 16)]
            ce = plsc.cumsum(e)
            cx = plsc.cumsum(e * x)
            cy = plsc.cumsum(e * y)
            bnd = vi != vi1
            scat(vi - g_lo, vi1 - g_lo, ce, cx, cy, bnd | last, bnd & notlast)
            return 0

        def vec_edge(j, _):
            ii = j * 16
            vi = iv[pl.ds(ii, 16)]
            vi1 = iv[pl.ds(ii + 1, 16)]
            posp = w0 + ii + lane
            valid = (posp >= c0) & (posp < e_hi)
            e = jnp.where(valid, jnp.exp(mv[pl.ds(ii, 16)]), 0.0)
            xb = 2 * ii - (ii % 128)
            x = pv[pl.ds(xb, 16)]
            y = pv[pl.ds(xb + 128, 16)]
            ce = plsc.cumsum(e)
            cx = plsc.cumsum(e * x)
            cy = plsc.cumsum(e * y)
            bnd = vi != vi1
            inr = (vi >= g_lo) & (vi < g_hi)
            inr1 = (vi1 >= g_lo) & (vi1 < g_hi)
            scat(vi - g_lo, vi1 - g_lo, ce, cx, cy,
                 (bnd | last) & inr, bnd & notlast & inr1)
            return 0

        @pl.when(clean)
        def _():
            lax.fori_loop(0, C // 16, vec_fast, 0, unroll=4)

        @pl.when(jnp.logical_not(clean))
        def _():
            lax.fori_loop(0, C // 16, vec_edge, 0, unroll=2)

    for i in range(NBUF - 1):
        @pl.when(i < n_chunks)
        def _(i=i):
            start_chunk(i, i)

    def outer(k2, _):
        for b in range(NBUF):
            k = NBUF * k2 + b

            @pl.when(k < n_chunks)
            def _(k=k, b=b):
                wait_chunk(b)

                @pl.when(k + NBUF - 1 < n_chunks)
                def _():
                    start_chunk(k + NBUF - 1, (b + NBUF - 1) % NBUF)

                process_chunk(k, b)
        return 0

    lax.fori_loop(0, (n_chunks + NBUF - 1) // NBUF, outer, 0)

    def fin_body(j, _):
        s = acc_e[pl.ds(j * 16, 16)]
        vx = acc_x[pl.ds(j * 16, 16)]
        vy = acc_y[pl.ds(j * 16, 16)]
        nz = s != 0.0
        den = jnp.where(nz, s, 1.0)
        cxo = jnp.where(nz, vx / den, 0.0)
        cyo = jnp.where(nz, vy / den, 0.0)
        oi = j * 32 + 2 * lane
        plsc.store_scatter(out_v, [oi], cxo)
        plsc.store_scatter(out_v, [oi + 1], cyo)
        return 0

    lax.fori_loop(0, SEGS // 16, fin_body, 0)
    pltpu.sync_copy(out_v, out_hbm.at[pl.ds(w * (2 * SEGS), 2 * SEGS)])


@functools.lru_cache(maxsize=None)
def _make_sc_call(N, G):
    SEGS = (-(-G // NW) + 15) // 16 * 16
    G_PAD = NW * SEGS
    C = CHUNK
    NBUF = NBUFS
    SAMP_PAD = (-(-N // SAMP) + 15) // 16 * 16
    mesh = plsc.VectorSubcoreMesh(core_axis_name="c", subcore_axis_name="s",
                                  num_cores=2, num_subcores=16)
    buf_scratch = []
    for _ in range(NBUF):
        buf_scratch += [
            pltpu.VMEM((C + 16,), jnp.int32),
            pltpu.VMEM((C,), jnp.float32),
            pltpu.VMEM((2 * C,), jnp.float32),
            pltpu.SemaphoreType.DMA,
        ]
    kern = pl.kernel(
        functools.partial(_sc_body, N, C, SEGS, NBUF, SAMP_PAD),
        out_type=jax.ShapeDtypeStruct((2 * G_PAD,), jnp.float32),
        mesh=mesh,
        compiler_params=pltpu.CompilerParams(needs_layout_passes=False),
        scratch_types=[pltpu.VMEM((SAMP_PAD,), jnp.int32)] + buf_scratch + [
            pltpu.VMEM((SEGS,), jnp.float32),
            pltpu.VMEM((SEGS,), jnp.float32),
            pltpu.VMEM((SEGS,), jnp.float32),
            pltpu.VMEM((2 * SEGS,), jnp.float32),
        ],
    )
    return kern, SAMP_PAD


def _tc_body(feat_ref, w_ref, u_ref, t_ref, b_ref, logit_ref, lzp_ref):
    x = feat_ref[...]
    wv = w_ref[...]
    acc = lax.dot_general(x, wv, (((1,), (0,)), ((), ())),
                          preferred_element_type=jnp.float32)
    acc = acc[:, :1]
    logit = 8.8 * jnp.tanh(acc + b_ref[0, 0])
    u = u_ref[...]
    noise = jnp.log(u) - jnp.log1p(-u)
    sl = (logit + noise) / t_ref[0, 0]
    logit_ref[...] = logit
    lzp_ref[...] = jax.nn.log_sigmoid(sl)


def kernel(glimpse__feature, glimpse_member__local_pos, glimpse_member__log_mask,
           glimpse_member__glimpse_index, temperature, W, b):
    G = glimpse__feature.shape[0]
    N = glimpse_member__glimpse_index.shape[0]

    idx = glimpse_member__glimpse_index.astype(jnp.int32)
    maskf = glimpse_member__log_mask.reshape(N)
    posb = (glimpse_member__local_pos
            .reshape(N // 128, 128, 2)
            .transpose(0, 2, 1)
            .reshape(2 * N))

    sc_call, samp_pad = _make_sc_call(N, G)
    n_samp = -(-N // SAMP)
    samp = jnp.concatenate(
        [idx[::SAMP],
         jnp.full((samp_pad - n_samp,), jnp.int32(2**31 - 1))])
    member_flat = sc_call(samp, idx, maskf, posb)
    member_center = member_flat[:2 * G].reshape(G, 2)

    u = jax.random.uniform(jax.random.key(42), (G,),
                           minval=1e-6, maxval=1.0 - 1e-6).reshape(G, 1)
    R = 2048
    t2 = temperature.reshape(1, 1)
    b2 = b.reshape(1, 1)
    wp = jnp.pad(W.T, ((0, 0), (0, 127)))
    logit2, lzp2 = pl.pallas_call(
        _tc_body,
        grid=(pl.cdiv(G, R),),
        in_specs=[
            pl.BlockSpec((R, 256), lambda i: (i, 0)),
            pl.BlockSpec((256, 128), lambda i: (0, 0)),
            pl.BlockSpec((R, 1), lambda i: (i, 0)),
            pl.BlockSpec((1, 1), lambda i: (0, 0)),
            pl.BlockSpec((1, 1), lambda i: (0, 0)),
        ],
        out_specs=[
            pl.BlockSpec((R, 1), lambda i: (i, 0)),
            pl.BlockSpec((R, 1), lambda i: (i, 0)),
        ],
        out_shape=[
            jax.ShapeDtypeStruct((G, 1), jnp.float32),
            jax.ShapeDtypeStruct((G, 1), jnp.float32),
        ],
    )(glimpse__feature, wp, u, t2, b2)

    return (lzp2[:, 0], logit2[:, 0], member_center)


# speedup vs baseline: 1.4668x; 1.4668x over previous
import functools

import jax
import jax.numpy as jnp
from jax import lax
from jax.experimental import pallas as pl
from jax.experimental.pallas import tpu as pltpu
from jax.experimental.pallas import tpu_sc as plsc

NW = 32
CHUNK = 4096
NBUFS = 4
SAMP = 4096


def _sc_body(N, C, SEGS, NBUF, SAMP_PAD, samp_hbm, idx_hbm, mask_hbm, pos_hbm,
             out_hbm, samp_v, *scr):
    bufs = tuple((scr[4 * i], scr[4 * i + 1], scr[4 * i + 2], scr[4 * i + 3])
                 for i in range(NBUF))
    acc_e, acc_x, acc_y, out_v = scr[4 * NBUF:4 * NBUF + 4]
    cid = lax.axis_index("c")
    sid = lax.axis_index("s")
    w = sid * 2 + cid
    lane = lax.iota(jnp.int32, 16)
    zf = jnp.zeros((16,), jnp.float32)

    def zero_body(j, _):
        acc_e[pl.ds(j * 16, 16)] = zf
        acc_x[pl.ds(j * 16, 16)] = zf
        acc_y[pl.ds(j * 16, 16)] = zf
        return 0

    lax.fori_loop(0, SEGS // 16, zero_body, 0)
    for i in range(NBUF):
        bufs[i][0][pl.ds(C, 16)] = jnp.zeros((16,), jnp.int32)

    g_lo = w * SEGS
    g_hi = g_lo + SEGS

    pltpu.sync_copy(samp_hbm, samp_v)
    zi = jnp.zeros((16,), jnp.int32)

    def cnt_body(j, carry):
        c1, c2 = carry
        v = samp_v[pl.ds(j * 16, 16)]
        c1 = c1 + jnp.where(v < g_lo, 1, 0)
        c2 = c2 + jnp.where(v < g_hi, 1, 0)
        return (c1, c2)

    c1v, c2v = lax.fori_loop(0, SAMP_PAD // 16, cnt_body, (zi, zi), unroll=4)
    cnt1 = jnp.sum(c1v)
    cnt2 = jnp.sum(c2v)
    a_lo = jnp.maximum(cnt1 - 1, 0) * SAMP
    e_lo = a_lo
    e_hi = jnp.minimum(cnt2 * SAMP, N)
    n_chunks = (e_hi - a_lo + (C - 1)) // C

    def chunk_w0(k):
        return pl.multiple_of(jnp.minimum(a_lo + k * C, N - C), 128)

    def start_chunk(k, b):
        iv, mv, pv, sem = bufs[b]
        w0 = chunk_w0(k)
        pltpu.async_copy(idx_hbm.at[pl.ds(w0, C)], iv.at[pl.ds(0, C)], sem)
        pltpu.async_copy(mask_hbm.at[pl.ds(w0, C)], mv, sem)
        pltpu.async_copy(pos_hbm.at[pl.ds(pl.multiple_of(2 * w0, 256), 2 * C)],
                         pv, sem)

    def wait_chunk(b):
        iv, mv, pv, sem = bufs[b]
        pltpu.make_async_copy(idx_hbm.at[pl.ds(0, C)], iv.at[pl.ds(0, C)],
                              sem).wait()
        pltpu.make_async_copy(mask_hbm.at[pl.ds(0, C)], mv, sem).wait()
        pltpu.make_async_copy(pos_hbm.at[pl.ds(0, 2 * C)], pv, sem).wait()

    last = lane == 15
    notlast = jnp.logical_not(last)

    def process_chunk(k, b):
        iv, mv, pv, _ = bufs[b]
        c0 = a_lo + k * C
        w0 = chunk_w0(k)
        clean = (c0 >= e_lo + SAMP) & (c0 + C <= e_hi - SAMP)

        def scat(li, li1, ce, cx, cy, m_p, m_m):
            plsc.addupdate_scatter(acc_e, [li], ce, mask=m_p)
            plsc.addupdate_scatter(acc_x, [li], cx, mask=m_p)
            plsc.addupdate_scatter(acc_y, [li], cy, mask=m_p)
            plsc.addupdate_scatter(acc_e, [li1], -ce, mask=m_m)
            plsc.addupdate_scatter(acc_x, [li1], -cx, mask=m_m)
            plsc.addupdate_scatter(acc_y, [li1], -cy, mask=m_m)

        def vec_fast(j, _):
            ii = j * 16
            vi = iv[pl.ds(ii, 16)]
            vi1 = iv[pl.ds(ii + 1, 16)]
            e = jnp.exp(mv[pl.ds(ii, 16)])
            xb = 2 * ii - (ii % 128)
            x = pv[pl.ds(xb, 16)]
            y = pv[pl.ds(xb + 128, 16)]
            ce = plsc.cumsum(e)
            cx = plsc.cumsum(e * x)
            cy = plsc.cumsum(e * y)
            bnd = vi != vi1
            scat(vi - g_lo, vi1 - g_lo, ce, cx, cy, bnd | last, bnd & notlast)
            return 0

        def vec_edge(j, _):
            ii = j * 16
            vi = iv[pl.ds(ii, 16)]
            vi1 = iv[pl.ds(ii + 1, 16)]
            posp = w0 + ii + lane
            valid = (posp >= c0) & (posp < e_hi)
            e = jnp.where(valid, jnp.exp(mv[pl.ds(ii, 16)]), 0.0)
            xb = 2 * ii - (ii % 128)
            x = pv[pl.ds(xb, 16)]
            y = pv[pl.ds(xb + 128, 16)]
            ce = plsc.cumsum(e)
            cx = plsc.cumsum(e * x)
            cy = plsc.cumsum(e * y)
            bnd = vi != vi1
            inr = (vi >= g_lo) & (vi < g_hi)
            inr1 = (vi1 >= g_lo) & (vi1 < g_hi)
            scat(vi - g_lo, vi1 - g_lo, ce, cx, cy,
                 (bnd | last) & inr, bnd & notlast & inr1)
            return 0

        @pl.when(clean)
        def _():
            lax.fori_loop(0, C // 16, vec_fast, 0, unroll=4)

        @pl.when(jnp.logical_not(clean))
        def _():
            lax.fori_loop(0, C // 16, vec_edge, 0, unroll=2)

    for i in range(NBUF - 1):
        @pl.when(i < n_chunks)
        def _(i=i):
            start_chunk(i, i)

    def outer(k2, _):
        for b in range(NBUF):
            k = NBUF * k2 + b

            @pl.when(k < n_chunks)
            def _(k=k, b=b):
                wait_chunk(b)

                @pl.when(k + NBUF - 1 < n_chunks)
                def _():
                    start_chunk(k + NBUF - 1, (b + NBUF - 1) % NBUF)

                process_chunk(k, b)
        return 0

    lax.fori_loop(0, (n_chunks + NBUF - 1) // NBUF, outer, 0)

    def fin_body(j, _):
        s = acc_e[pl.ds(j * 16, 16)]
        vx = acc_x[pl.ds(j * 16, 16)]
        vy = acc_y[pl.ds(j * 16, 16)]
        nz = s != 0.0
        den = jnp.where(nz, s, 1.0)
        cxo = jnp.where(nz, vx / den, 0.0)
        cyo = jnp.where(nz, vy / den, 0.0)
        xo = 256 * (j // 8) + 16 * (j % 8)
        out_v[pl.ds(xo, 16)] = cxo
        out_v[pl.ds(xo + 128, 16)] = cyo
        return 0

    lax.fori_loop(0, SEGS // 16, fin_body, 0)
    pltpu.sync_copy(out_v, out_hbm.at[pl.ds(w * (2 * SEGS), 2 * SEGS)])


@functools.lru_cache(maxsize=None)
def _make_sc_call(N, G):
    SEGS = (-(-G // NW) + 127) // 128 * 128
    G_PAD = NW * SEGS
    C = CHUNK
    NBUF = NBUFS
    SAMP_PAD = (-(-N // SAMP) + 15) // 16 * 16
    mesh = plsc.VectorSubcoreMesh(core_axis_name="c", subcore_axis_name="s",
                                  num_cores=2, num_subcores=16)
    buf_scratch = []
    for _ in range(NBUF):
        buf_scratch += [
            pltpu.VMEM((C + 16,), jnp.int32),
            pltpu.VMEM((C,), jnp.float32),
            pltpu.VMEM((2 * C,), jnp.float32),
            pltpu.SemaphoreType.DMA,
        ]
    kern = pl.kernel(
        functools.partial(_sc_body, N, C, SEGS, NBUF, SAMP_PAD),
        out_type=jax.ShapeDtypeStruct((2 * G_PAD,), jnp.float32),
        mesh=mesh,
        compiler_params=pltpu.CompilerParams(needs_layout_passes=False),
        scratch_types=[pltpu.VMEM((SAMP_PAD,), jnp.int32)] + buf_scratch + [
            pltpu.VMEM((SEGS,), jnp.float32),
            pltpu.VMEM((SEGS,), jnp.float32),
            pltpu.VMEM((SEGS,), jnp.float32),
            pltpu.VMEM((2 * SEGS,), jnp.float32),
        ],
    )
    return kern, SAMP_PAD


def _tc_body(R, feat_ref, w_ref, u_ref, t_ref, b_ref, logit_ref, lzp_ref):
    x = feat_ref[...]
    wv = w_ref[...]
    acc = lax.dot_general(x, wv, (((1,), (0,)), ((), ())),
                          preferred_element_type=jnp.float32)
    acc = acc[:, :1].reshape(R // 128, 128)
    logit = 8.8 * jnp.tanh(acc + b_ref[0, 0])
    u = u_ref[...]
    noise = jnp.log(u) - jnp.log1p(-u)
    sl = (logit + noise) / t_ref[0, 0]
    logit_ref[...] = logit
    lzp_ref[...] = jax.nn.log_sigmoid(sl)


def kernel(glimpse__feature, glimpse_member__local_pos, glimpse_member__log_mask,
           glimpse_member__glimpse_index, temperature, W, b):
    G = glimpse__feature.shape[0]
    N = glimpse_member__glimpse_index.shape[0]

    idx = glimpse_member__glimpse_index.astype(jnp.int32)
    maskf = glimpse_member__log_mask.reshape(N)
    posb = (glimpse_member__local_pos
            .reshape(N // 128, 128, 2)
            .transpose(0, 2, 1)
            .reshape(2 * N))

    sc_call, samp_pad = _make_sc_call(N, G)
    n_samp = -(-N // SAMP)
    samp = jnp.concatenate(
        [idx[::SAMP],
         jnp.full((samp_pad - n_samp,), jnp.int32(2**31 - 1))])
    member_flat = sc_call(samp, idx, maskf, posb)
    g_pad = member_flat.shape[0] // 2
    member_center = (member_flat
                     .reshape(g_pad // 128, 2, 128)
                     .transpose(0, 2, 1)
                     .reshape(g_pad, 2)[:G])

    R = 2048
    g_up = pl.cdiv(G, R) * R
    u = jax.random.uniform(jax.random.key(42), (G,),
                           minval=1e-6, maxval=1.0 - 1e-6)
    u2d = jnp.concatenate(
        [u, jnp.full((g_up - G,), 0.5, jnp.float32)]).reshape(g_up // 128, 128)
    t2 = temperature.reshape(1, 1)
    b2 = b.reshape(1, 1)
    wp = jnp.pad(W.T, ((0, 0), (0, 127)))
    logit2, lzp2 = pl.pallas_call(
        functools.partial(_tc_body, R),
        grid=(g_up // R,),
        in_specs=[
            pl.BlockSpec((R, 256), lambda i: (i, 0)),
            pl.BlockSpec((256, 128), lambda i: (0, 0)),
            pl.BlockSpec((R // 128, 128), lambda i: (i, 0)),
            pl.BlockSpec((1, 1), lambda i: (0, 0)),
            pl.BlockSpec((1, 1), lambda i: (0, 0)),
        ],
        out_specs=[
            pl.BlockSpec((R // 128, 128), lambda i: (i, 0)),
            pl.BlockSpec((R // 128, 128), lambda i: (i, 0)),
        ],
        out_shape=[
            jax.ShapeDtypeStruct((g_up // 128, 128), jnp.float32),
            jax.ShapeDtypeStruct((g_up // 128, 128), jnp.float32),
        ],
    )(glimpse__feature, wp, u2d, t2, b2)

    return (lzp2.reshape(g_up)[:G], logit2.reshape(g_up)[:G], member_center)
